# Initial kernel scaffold; baseline (speedup 1.0000x reference)
import jax
import jax.numpy as jnp
from jax import lax
from jax.experimental import pallas as pl
from jax.experimental.pallas import tpu as pltpu
from jax.experimental.pallas import tpu_sc as plsc

N = 100000
E = 3200000
NC = 2
NS = 16
NW = NC * NS
CHUNK = 128
CB = 16
BLK_E = CB * CHUNK
NBLK = -(-E // (NW * BLK_E))
EW = NBLK * BLK_E
E_PAD = NW * EW
ACC_R = N + 16
ZR = ACC_R // NS
WR = N // NS


def _sc_aggregate_body(t_hbm, s_hbm, d_hbm, z_hbm, out_hbm,
                       sidx, didx, rows, acc, gsem, ssem):
    cid = lax.axis_index("c")
    sid = lax.axis_index("s")
    wid = sid * NC + cid

    pltpu.sync_copy(z_hbm.at[pl.ds(sid * ZR, ZR)], acc.at[pl.ds(sid * ZR, ZR)])
    plsc.subcore_barrier()

    wrow0 = wid * (NBLK * CB)

    def blk(b, carry):
        row0 = wrow0 + b * CB
        pltpu.sync_copy(s_hbm.at[pl.ds(row0, CB)], sidx)
        pltpu.sync_copy(d_hbm.at[pl.ds(row0, CB)], didx)
        g = [pltpu.async_copy(t_hbm.at[sidx.at[j]], rows.at[j], gsem)
             for j in range(CB)]
        for h in g:
            h.wait()
        s = [pltpu.async_copy(rows.at[j], acc.at[didx.at[j]], ssem, add=True)
             for j in range(CB)]
        for h in s:
            h.wait()
        return carry

    lax.fori_loop(0, NBLK, blk, 0)

    plsc.subcore_barrier()
    pltpu.sync_copy(acc.at[pl.ds(sid * WR, WR)],
                    out_hbm.at[cid].at[pl.ds(sid * WR, WR)])


def _sc_aggregate(table, src2d, dst2d, zeros_hbm):
    mesh = plsc.VectorSubcoreMesh(core_axis_name="c", subcore_axis_name="s")
    f = pl.kernel(
        _sc_aggregate_body,
        out_type=jax.ShapeDtypeStruct((NC, N, 16), jnp.float32),
        mesh=mesh,
        scratch_types=[
            pltpu.VMEM((CB, CHUNK), jnp.int32),
            pltpu.VMEM((CB, CHUNK), jnp.int32),
            pltpu.VMEM((CB, CHUNK, 16), jnp.float32),
            pltpu.VMEM_SHARED((ACC_R, 16), jnp.float32),
            pltpu.SemaphoreType.DMA,
            pltpu.SemaphoreType.DMA,
        ],
    )
    return f(table, src2d, dst2d, zeros_hbm)


def _tc_layer1_body(p_ref, t_ref, wl_ref, wr_ref, b_ref, h_ref):
    s = p_ref[0] + p_ref[1]
    inv = 1.0 / jnp.maximum(s[:, 8:9], 1.0)
    mean = s * inv
    h = (jnp.dot(mean, wl_ref[...], preferred_element_type=jnp.float32)
         + jnp.dot(t_ref[...], wr_ref[...], preferred_element_type=jnp.float32)
         + b_ref[...])
    h_ref[...] = jnp.maximum(h, 0.0)


def _tc_layer2_body(p2_ref, p1_ref, h_ref, wl_ref, wr_ref, b_ref, o_ref):
    cnt = p1_ref[0][:, 8:9] + p1_ref[1][:, 8:9]
    inv = 1.0 / jnp.maximum(cnt, 1.0)
    s = p2_ref[0] + p2_ref[1]
    o_ref[...] = (jnp.dot(s * inv, wl_ref[...], preferred_element_type=jnp.float32)
                  + jnp.dot(h_ref[...], wr_ref[...], preferred_element_type=jnp.float32)
                  + b_ref[...])


_BN = 5000


def _tc_layer1(p1, table1, W1l_pad, W1r_pad, b1):
    return pl.pallas_call(
        _tc_layer1_body,
        grid=(N // _BN,),
        in_specs=[
            pl.BlockSpec((NC, _BN, 16), lambda i: (0, i, 0)),
            pl.BlockSpec((_BN, 16), lambda i: (i, 0)),
            pl.BlockSpec((16, 16), lambda i: (0, 0)),
            pl.BlockSpec((16, 16), lambda i: (0, 0)),
            pl.BlockSpec((1, 16), lambda i: (0, 0)),
        ],
        out_specs=pl.BlockSpec((_BN, 16), lambda i: (i, 0)),
        out_shape=jax.ShapeDtypeStruct((N, 16), jnp.float32),
    )(p1, table1, W1l_pad, W1r_pad, b1)


def _tc_layer2(p2, p1, h, W2_l, W2_r, b2):
    return pl.pallas_call(
        _tc_layer2_body,
        grid=(N // _BN,),
        in_specs=[
            pl.BlockSpec((NC, _BN, 16), lambda i: (0, i, 0)),
            pl.BlockSpec((NC, _BN, 16), lambda i: (0, i, 0)),
            pl.BlockSpec((_BN, 16), lambda i: (i, 0)),
            pl.BlockSpec((16, 16), lambda i: (0, 0)),
            pl.BlockSpec((16, 16), lambda i: (0, 0)),
            pl.BlockSpec((1, 16), lambda i: (0, 0)),
        ],
        out_specs=pl.BlockSpec((_BN, 16), lambda i: (i, 0)),
        out_shape=jax.ShapeDtypeStruct((N, 16), jnp.float32),
    )(p2, p1, h, W2_l, W2_r, b2)


def kernel(x, edge_index, W1_l, b1, W1_r, W2_l, b2, W2_r):
    src = edge_index[0]
    dst = edge_index[1]
    pad = E_PAD - E
    src_p = jnp.concatenate([src, jnp.zeros((pad,), jnp.int32)])
    dst_p = jnp.concatenate([dst, jnp.full((pad,), N, jnp.int32)])
    src2d = src_p.reshape(E_PAD // CHUNK, CHUNK)
    dst2d = dst_p.reshape(E_PAD // CHUNK, CHUNK)
    zeros_hbm = jnp.zeros((ACC_R, 16), jnp.float32)

    table1 = jnp.concatenate(
        [x, jnp.ones((N, 1), jnp.float32), jnp.zeros((N, 7), jnp.float32)], axis=1)
    W1l_pad = jnp.concatenate([W1_l, jnp.zeros((8, 16), jnp.float32)], axis=0)
    W1r_pad = jnp.concatenate([W1_r, jnp.zeros((8, 16), jnp.float32)], axis=0)

    p1 = _sc_aggregate(table1, src2d, dst2d, zeros_hbm)
    h = _tc_layer1(p1, table1, W1l_pad, W1r_pad, b1.reshape(1, 16))
    p2 = _sc_aggregate(h, src2d, dst2d, zeros_hbm)
    out = _tc_layer2(p2, p1, h, W2_l, W2_r, b2.reshape(1, 16))
    return out


# speedup vs baseline: 33.9003x; 33.9003x over previous
import jax
import jax.numpy as jnp
from jax import lax
from jax.experimental import pallas as pl
from jax.experimental.pallas import tpu as pltpu
from jax.experimental.pallas import tpu_sc as plsc

N = 100000
E = 3200000
NC = 2
NS = 16
NW = NC * NS
CHUNK = 128
CB = 8
BLK_E = CB * CHUNK
NBLK = -(-E // (NW * BLK_E))
EW = NBLK * BLK_E
E_PAD = NW * EW
ACC_R = 100096
ZR = ACC_R // NS


def _sc_aggregate_body(t_hbm, s_hbm, d_hbm, z_hbm, out_hbm,
                       sidx, didx, rows, acc, gsem, ssem):
    cid = lax.axis_index("c")
    sid = lax.axis_index("s")
    wid = sid * NC + cid

    pltpu.sync_copy(z_hbm.at[pl.ds(sid * ZR, ZR)], acc.at[pl.ds(sid * ZR, ZR)])
    plsc.subcore_barrier()

    wrow0 = wid * (NBLK * CB)

    def blk(b, carry):
        row0 = wrow0 + b * CB
        pltpu.sync_copy(s_hbm.at[pl.ds(row0, CB)], sidx)
        pltpu.sync_copy(d_hbm.at[pl.ds(row0, CB)], didx)
        g = [pltpu.async_copy(t_hbm.at[sidx.at[j]], rows.at[j], gsem)
             for j in range(CB)]
        for h in g:
            h.wait()
        s = [pltpu.async_copy(rows.at[j], acc.at[didx.at[j]], ssem, add=True)
             for j in range(CB)]
        for h in s:
            h.wait()
        return carry

    lax.fori_loop(0, NBLK, blk, 0)

    plsc.subcore_barrier()
    pltpu.sync_copy(acc.at[pl.ds(sid * ZR, ZR)],
                    out_hbm.at[cid].at[pl.ds(sid * ZR, ZR)])


def _sc_aggregate(table, src2d, dst2d, zeros_hbm):
    mesh = plsc.VectorSubcoreMesh(core_axis_name="c", subcore_axis_name="s")
    f = pl.kernel(
        _sc_aggregate_body,
        out_type=jax.ShapeDtypeStruct((NC, ACC_R, 16), jnp.float32),
        mesh=mesh,
        compiler_params=pltpu.CompilerParams(use_tc_tiling_on_sc=False),
        scratch_types=[
            pltpu.VMEM((CB, CHUNK), jnp.int32),
            pltpu.VMEM((CB, CHUNK), jnp.int32),
            pltpu.VMEM((CB, CHUNK, 16), jnp.float32),
            pltpu.VMEM_SHARED((ACC_R, 16), jnp.float32),
            pltpu.SemaphoreType.DMA,
            pltpu.SemaphoreType.DMA,
        ],
    )
    return f(table, src2d, dst2d, zeros_hbm)


def _tc_layer1_body(p_ref, t_ref, wl_ref, wr_ref, b_ref, h_ref):
    s = p_ref[0] + p_ref[1]
    inv = 1.0 / jnp.maximum(s[:, 8:9], 1.0)
    mean = s * inv
    h = (jnp.dot(mean, wl_ref[...], preferred_element_type=jnp.float32)
         + jnp.dot(t_ref[...], wr_ref[...], preferred_element_type=jnp.float32)
         + b_ref[...])
    h_ref[...] = jnp.maximum(h, 0.0)


def _tc_layer2_body(p2_ref, p1_ref, h_ref, wl_ref, wr_ref, b_ref, o_ref):
    cnt = p1_ref[0][:, 8:9] + p1_ref[1][:, 8:9]
    inv = 1.0 / jnp.maximum(cnt, 1.0)
    s = p2_ref[0] + p2_ref[1]
    o_ref[...] = (jnp.dot(s * inv, wl_ref[...], preferred_element_type=jnp.float32)
                  + jnp.dot(h_ref[...], wr_ref[...], preferred_element_type=jnp.float32)
                  + b_ref[...])


_BN = 6256


def _tc_layer1(p1, table1, W1l_pad, W1r_pad, b1):
    return pl.pallas_call(
        _tc_layer1_body,
        grid=(ACC_R // _BN,),
        in_specs=[
            pl.BlockSpec((NC, _BN, 16), lambda i: (0, i, 0)),
            pl.BlockSpec((_BN, 16), lambda i: (i, 0)),
            pl.BlockSpec((16, 16), lambda i: (0, 0)),
            pl.BlockSpec((16, 16), lambda i: (0, 0)),
            pl.BlockSpec((1, 16), lambda i: (0, 0)),
        ],
        out_specs=pl.BlockSpec((_BN, 16), lambda i: (i, 0)),
        out_shape=jax.ShapeDtypeStruct((ACC_R, 16), jnp.float32),
    )(p1, table1, W1l_pad, W1r_pad, b1)


def _tc_layer2(p2, p1, h, W2_l, W2_r, b2):
    return pl.pallas_call(
        _tc_layer2_body,
        grid=(ACC_R // _BN,),
        in_specs=[
            pl.BlockSpec((NC, _BN, 16), lambda i: (0, i, 0)),
            pl.BlockSpec((NC, _BN, 16), lambda i: (0, i, 0)),
            pl.BlockSpec((_BN, 16), lambda i: (i, 0)),
            pl.BlockSpec((16, 16), lambda i: (0, 0)),
            pl.BlockSpec((16, 16), lambda i: (0, 0)),
            pl.BlockSpec((1, 16), lambda i: (0, 0)),
        ],
        out_specs=pl.BlockSpec((_BN, 16), lambda i: (i, 0)),
        out_shape=jax.ShapeDtypeStruct((ACC_R, 16), jnp.float32),
    )(p2, p1, h, W2_l, W2_r, b2)


def kernel(x, edge_index, W1_l, b1, W1_r, W2_l, b2, W2_r):
    src = edge_index[0]
    dst = edge_index[1]
    pad = E_PAD - E
    src_p = jnp.concatenate([src, jnp.zeros((pad,), jnp.int32)])
    dst_p = jnp.concatenate([dst, jnp.full((pad,), N, jnp.int32)])
    src2d = src_p.reshape(E_PAD // CHUNK, CHUNK)
    dst2d = dst_p.reshape(E_PAD // CHUNK, CHUNK)
    zeros_hbm = jnp.zeros((ACC_R, 16), jnp.float32)

    table1 = jnp.concatenate(
        [x, jnp.ones((N, 1), jnp.float32), jnp.zeros((N, 7), jnp.float32)], axis=1)
    table1 = jnp.pad(table1, ((0, ACC_R - N), (0, 0)))
    W1l_pad = jnp.concatenate([W1_l, jnp.zeros((8, 16), jnp.float32)], axis=0)
    W1r_pad = jnp.concatenate([W1_r, jnp.zeros((8, 16), jnp.float32)], axis=0)

    p1 = _sc_aggregate(table1, src2d, dst2d, zeros_hbm)
    h = _tc_layer1(p1, table1, W1l_pad, W1r_pad, b1.reshape(1, 16))
    p2 = _sc_aggregate(h, src2d, dst2d, zeros_hbm)
    out = _tc_layer2(p2, p1, h, W2_l, W2_r, b2.reshape(1, 16))
    return out[:N]
